# baseline (speedup 1.0000x reference)
import functools

import jax
import jax.numpy as jnp
from jax import lax
from jax.experimental import pallas as pl
from jax.experimental.pallas import tpu as pltpu
from jax.experimental.pallas import tpu_sc as plsc

_HID = 64
_EMB = 32
_K = 1024


def _enc1_body(p_ref, w_ref, b_ref, o_ref):
    p = p_ref[0].reshape(112 * 112, 16)
    y = jnp.dot(p, w_ref[...], preferred_element_type=jnp.float32) + b_ref[0]
    o_ref[0] = jnp.maximum(y, 0.0).reshape(112, 112, _HID)


def _enc2_body(e_ref, w_ref, b_ref, o_ref):
    acc = jnp.zeros((56 * 56, _EMB), jnp.float32)
    for i in range(2):
        for j in range(2):
            blk = e_ref[0, i:i + 56, j:j + 56, :].reshape(56 * 56, 4 * _HID)
            acc = acc + jnp.dot(blk, w_ref[i * 2 + j],
                                preferred_element_type=jnp.float32)
    o_ref[0] = (acc + b_ref[0]).reshape(56, 56, _EMB)


def _vq_body(z_ref, cbt_ref, idx_ref):
    zf = z_ref[0]
    cbt = cbt_ref[...]
    z2 = jnp.sum(zf * zf, axis=1, keepdims=True)
    c2 = jnp.sum(cbt * cbt, axis=0)
    mm = jnp.dot(zf, cbt, preferred_element_type=jnp.float32)
    d = (z2 + c2[None, :]) - 2.0 * mm
    dmin = jnp.min(d, axis=1, keepdims=True)
    ii = lax.broadcasted_iota(jnp.int32, d.shape, 1)
    idx_ref[0, 0] = jnp.min(jnp.where(d == dmin, ii, _K), axis=1)


def _qst_body(z_ref, q_ref, qst_ref, ls_ref):
    z = z_ref[...]
    q = q_ref[...]
    qst_ref[...] = z + (q - z)
    df = q - z
    ls_ref[0, 0] = jnp.sum(df * df)


def _dec1_body(x_ref, w_ref, b_ref, o_ref):
    b = b_ref[0]
    for py in range(2):
        for px in range(2):
            acc = jnp.zeros((56 * 56, _HID), jnp.float32)
            for i in range(2):
                for j in range(2):
                    blk = x_ref[0, py + i:py + i + 56, px + j:px + j + 56, :]
                    blk = blk.reshape(56 * 56, _EMB)
                    acc = acc + jnp.dot(
                        blk, w_ref[((py * 2 + px) * 2 + i) * 2 + j],
                        preferred_element_type=jnp.float32)
            o_ref[py * 2 + px, 0] = jnp.maximum(acc + b, 0.0).reshape(56, 56, _HID)


def _dec2_body(y_ref, w_ref, b_ref, o_ref):
    b = b_ref[0, 0]
    for py in range(2):
        for px in range(2):
            acc = jnp.zeros((112, 112), jnp.float32)
            for i in range(2):
                for j in range(2):
                    blk = y_ref[0, py + i:py + i + 112, px + j:px + j + 112, :]
                    wv = w_ref[((py * 2 + px) * 2 + i) * 2 + j]
                    acc = acc + jnp.sum(blk * wv[None, None, :], axis=-1)
            o_ref[py * 2 + px, 0] = jax.nn.sigmoid(acc + b)


def _vq_gather(codebook, idx_flat):
    info = plsc.get_sparse_core_info()
    nw = info.num_cores * info.num_subcores
    n = idx_flat.shape[0]
    bpw = n // nw
    nchunk = 2
    cpw = bpw // nchunk
    cb128 = jnp.pad(codebook, ((0, 0), (0, 128 - _EMB)))
    mesh = plsc.VectorSubcoreMesh(core_axis_name="c", subcore_axis_name="s")

    @functools.partial(
        pl.kernel, mesh=mesh,
        out_type=jax.ShapeDtypeStruct((n, 128), jnp.float32),
        scratch_types=[
            pltpu.VMEM((bpw,), jnp.int32),
            pltpu.VMEM((cpw, 128), jnp.float32),
            pltpu.SemaphoreType.DMA,
        ],
    )
    def gather_k(cb_hbm, idx_hbm, out_hbm, idx_v, rows_v, sem):
        wid = lax.axis_index("s") * info.num_cores + lax.axis_index("c")
        base = wid * bpw
        pltpu.sync_copy(idx_hbm.at[pl.ds(base, bpw)], idx_v)
        for c in range(nchunk):
            pltpu.async_copy(
                cb_hbm.at[idx_v.at[pl.ds(c * cpw, cpw)]], rows_v, sem).wait()
            pltpu.sync_copy(rows_v, out_hbm.at[pl.ds(base + c * cpw, cpw)])

    return gather_k(cb128, idx_flat)[:, :_EMB]


def kernel(x, enc1_w, enc1_b, enc2_w, enc2_b, codebook,
           dec1_w, dec1_b, dec2_w, dec2_b):
    f32 = jnp.float32
    n = x.shape[0]

    xp = jnp.pad(x[:, 0], ((0, 0), (1, 1), (1, 1)))
    p16 = jnp.stack([xp[:, kh:kh + 224:2, kw:kw + 224:2]
                     for kh in range(4) for kw in range(4)], axis=-1)
    w1 = enc1_w[:, 0].reshape(_HID, 16).T
    z1 = pl.pallas_call(
        _enc1_body,
        grid=(n,),
        in_specs=[
            pl.BlockSpec((1, 112, 112, 16), lambda b: (b, 0, 0, 0)),
            pl.BlockSpec((16, _HID), lambda b: (0, 0)),
            pl.BlockSpec((1, _HID), lambda b: (0, 0)),
        ],
        out_specs=pl.BlockSpec((1, 112, 112, _HID), lambda b: (b, 0, 0, 0)),
        out_shape=jax.ShapeDtypeStruct((n, 112, 112, _HID), f32),
    )(p16, w1, enc1_b.reshape(1, _HID))

    z1p = jnp.pad(z1, ((0, 0), (1, 1), (1, 1), (0, 0)))
    e = (z1p.reshape(n, 57, 2, 57, 2, _HID)
         .transpose(0, 1, 3, 2, 4, 5)
         .reshape(n, 57, 57, 4 * _HID))
    w2 = jnp.stack([
        jnp.concatenate([enc2_w[:, :, a + 2 * i, b + 2 * j].T
                         for a in range(2) for b in range(2)], axis=0)
        for i in range(2) for j in range(2)], axis=0)
    z = pl.pallas_call(
        _enc2_body,
        grid=(n,),
        in_specs=[
            pl.BlockSpec((1, 57, 57, 4 * _HID), lambda b: (b, 0, 0, 0)),
            pl.BlockSpec((4, 4 * _HID, _EMB), lambda b: (0, 0, 0)),
            pl.BlockSpec((1, _EMB), lambda b: (0, 0)),
        ],
        out_specs=pl.BlockSpec((1, 56, 56, _EMB), lambda b: (b, 0, 0, 0)),
        out_shape=jax.ShapeDtypeStruct((n, 56, 56, _EMB), f32),
    )(e, w2, enc2_b.reshape(1, _EMB))

    nrows = n * 56 * 56
    zf3 = z.reshape(nrows // 1024, 1024, _EMB)
    idx3 = pl.pallas_call(
        _vq_body,
        grid=(nrows // 1024,),
        in_specs=[
            pl.BlockSpec((1, 1024, _EMB), lambda b: (b, 0, 0)),
            pl.BlockSpec((_EMB, _K), lambda b: (0, 0)),
        ],
        out_specs=pl.BlockSpec((1, 1, 1024), lambda b: (b, 0, 0)),
        out_shape=jax.ShapeDtypeStruct((nrows // 1024, 1, 1024), jnp.int32),
    )(zf3, codebook.T)
    idx_flat = idx3.reshape(nrows)

    q = _vq_gather(codebook, idx_flat)

    nel = nrows * _EMB
    zflat = z.reshape(nel // 128, 128)
    qflat = q.reshape(nel // 128, 128)
    qst2, ls = pl.pallas_call(
        _qst_body,
        out_shape=[
            jax.ShapeDtypeStruct((nel // 128, 128), f32),
            jax.ShapeDtypeStruct((1, 1), f32),
        ],
        out_specs=[
            pl.BlockSpec(memory_space=pltpu.VMEM),
            pl.BlockSpec(memory_space=pltpu.SMEM),
        ],
    )(zflat, qflat)
    m = ls[0, 0] / float(nel)
    vq_loss = m + 0.25 * m

    qstp = jnp.pad(qst2.reshape(n, 56, 56, _EMB),
                   ((0, 0), (1, 1), (1, 1), (0, 0)))
    w3 = jnp.stack([dec1_w[:, :, 2 * i + py, 2 * j + px].T
                    for py in range(2) for px in range(2)
                    for i in range(2) for j in range(2)], axis=0)
    y_ph = pl.pallas_call(
        _dec1_body,
        grid=(n,),
        in_specs=[
            pl.BlockSpec((1, 58, 58, _EMB), lambda b: (b, 0, 0, 0)),
            pl.BlockSpec((16, _EMB, _HID), lambda b: (0, 0, 0)),
            pl.BlockSpec((1, _HID), lambda b: (0, 0)),
        ],
        out_specs=pl.BlockSpec((4, 1, 56, 56, _HID), lambda b: (0, b, 0, 0, 0)),
        out_shape=jax.ShapeDtypeStruct((4, n, 56, 56, _HID), f32),
    )(qstp, w3, dec1_b.reshape(1, _HID))
    y1 = (y_ph.reshape(2, 2, n, 56, 56, _HID)
          .transpose(2, 3, 0, 4, 1, 5)
          .reshape(n, 112, 112, _HID))

    y1p = jnp.pad(y1, ((0, 0), (1, 1), (1, 1), (0, 0)))
    w4 = jnp.stack([dec2_w[0, :, 2 * i + py, 2 * j + px]
                    for py in range(2) for px in range(2)
                    for i in range(2) for j in range(2)], axis=0)
    r_ph = pl.pallas_call(
        _dec2_body,
        grid=(n,),
        in_specs=[
            pl.BlockSpec((1, 114, 114, _HID), lambda b: (b, 0, 0, 0)),
            pl.BlockSpec((16, _HID), lambda b: (0, 0)),
            pl.BlockSpec((1, 1), lambda b: (0, 0), memory_space=pltpu.SMEM),
        ],
        out_specs=pl.BlockSpec((4, 1, 112, 112), lambda b: (0, b, 0, 0)),
        out_shape=jax.ShapeDtypeStruct((4, n, 112, 112), f32),
    )(y1p, w4, dec2_b.reshape(1, 1))
    x_recon = (r_ph.reshape(2, 2, n, 112, 112)
               .transpose(2, 3, 0, 4, 1)
               .reshape(n, 1, 224, 224))

    indices = idx_flat.reshape(n, 56, 56)
    return (x_recon, vq_loss, indices)


# speedup vs baseline: 1.0949x; 1.0949x over previous
import functools

import jax
import jax.numpy as jnp
from jax import lax
from jax.experimental import pallas as pl
from jax.experimental.pallas import tpu as pltpu
from jax.experimental.pallas import tpu_sc as plsc

_HID = 64
_EMB = 32
_K = 1024


def _enc1_body(p_ref, w_ref, b_ref, o_ref):
    p = p_ref[0].reshape(112 * 112, 16)
    y = jnp.dot(p, w_ref[...], preferred_element_type=jnp.float32) + b_ref[0]
    o_ref[0] = jnp.maximum(y, 0.0).reshape(112, 112, _HID)


def _enc2_body(e_ref, w_ref, b_ref, o_ref):
    acc = jnp.zeros((56 * 56, _EMB), jnp.float32)
    for i in range(2):
        for j in range(2):
            blk = e_ref[0, i:i + 56, j:j + 56, :].reshape(56 * 56, 4 * _HID)
            acc = acc + jnp.dot(blk, w_ref[i * 2 + j],
                                preferred_element_type=jnp.float32)
    o_ref[0] = (acc + b_ref[0]).reshape(56, 56, _EMB)


def _vq_body(z_ref, cbt_ref, idx_ref):
    zf = z_ref[0]
    cbt = cbt_ref[...]
    z2 = jnp.sum(zf * zf, axis=1, keepdims=True)
    c2 = jnp.sum(cbt * cbt, axis=0)
    mm = jnp.dot(zf, cbt, preferred_element_type=jnp.float32)
    d = (z2 + c2[None, :]) - 2.0 * mm
    dmin = jnp.min(d, axis=1, keepdims=True)
    ii = lax.broadcasted_iota(jnp.int32, d.shape, 1)
    idx_ref[0, 0] = jnp.min(jnp.where(d == dmin, ii, _K), axis=1)


def _dec1_body(z_ref, q_ref, w_ref, b_ref, o_ref, ls_ref):
    z = z_ref[0]
    q = q_ref[0][:, :, :_EMB]
    df = q - z
    qst = z + df

    @pl.when(pl.program_id(0) == 0)
    def _():
        ls_ref[0, 0] = 0.0
    ls_ref[0, 0] += jnp.sum(df * df)

    zc = jnp.zeros((1, 56, _EMB), jnp.float32)
    t = jnp.concatenate([zc, qst, zc], axis=0)
    zr = jnp.zeros((58, 1, _EMB), jnp.float32)
    xp_ = jnp.concatenate([zr, t, zr], axis=1)
    b = b_ref[0]
    for py in range(2):
        for px in range(2):
            acc = jnp.zeros((56 * 56, _HID), jnp.float32)
            for i in range(2):
                for j in range(2):
                    blk = xp_[py + i:py + i + 56, px + j:px + j + 56, :]
                    blk = blk.reshape(56 * 56, _EMB)
                    acc = acc + jnp.dot(
                        blk, w_ref[((py * 2 + px) * 2 + i) * 2 + j],
                        preferred_element_type=jnp.float32)
            o_ref[py * 2 + px, 0] = jnp.maximum(acc + b, 0.0).reshape(56, 56, _HID)


def _dec2_body(y_ref, w_ref, b_ref, o_ref):
    b = b_ref[0, 0]
    acc = jnp.zeros((112 * 112, 4), jnp.float32)
    for u in range(3):
        for v in range(3):
            blk = y_ref[0, u:u + 112, v:v + 112, :].reshape(112 * 112, _HID)
            acc = acc + jnp.dot(blk, w_ref[u * 3 + v],
                                preferred_element_type=jnp.float32)
    o_ref[0] = jax.nn.sigmoid(acc + b).reshape(112, 112, 4)


def _vq_gather(codebook, idx_flat):
    info = plsc.get_sparse_core_info()
    nw = info.num_cores * info.num_subcores
    n = idx_flat.shape[0]
    bpw = n // nw
    nchunk = 4
    cpw = bpw // nchunk
    cb128 = jnp.pad(codebook, ((0, 0), (0, 128 - _EMB)))
    mesh = plsc.VectorSubcoreMesh(core_axis_name="c", subcore_axis_name="s")

    @functools.partial(
        pl.kernel, mesh=mesh,
        out_type=jax.ShapeDtypeStruct((n, 128), jnp.float32),
        scratch_types=[
            pltpu.VMEM((bpw,), jnp.int32),
            pltpu.VMEM((cpw, 128), jnp.float32),
            pltpu.VMEM((cpw, 128), jnp.float32),
            pltpu.SemaphoreType.DMA,
            pltpu.SemaphoreType.DMA,
        ],
    )
    def gather_k(cb_hbm, idx_hbm, out_hbm, idx_v, rows_a, rows_b, sem_a, sem_b):
        wid = lax.axis_index("s") * info.num_cores + lax.axis_index("c")
        base = wid * bpw
        pltpu.sync_copy(idx_hbm.at[pl.ds(base, bpw)], idx_v)
        bufs = (rows_a, rows_b)
        sems = (sem_a, sem_b)
        cps = [None, None]
        for c in range(nchunk):
            cps[c % 2] = pltpu.async_copy(
                cb_hbm.at[idx_v.at[pl.ds(c * cpw, cpw)]], bufs[c % 2],
                sems[c % 2])
            if c >= 1:
                w = c - 1
                cps[w % 2].wait()
                pltpu.sync_copy(bufs[w % 2],
                                out_hbm.at[pl.ds(base + w * cpw, cpw)])
        cps[(nchunk - 1) % 2].wait()
        pltpu.sync_copy(bufs[(nchunk - 1) % 2],
                        out_hbm.at[pl.ds(base + (nchunk - 1) * cpw, cpw)])

    return gather_k(cb128, idx_flat)


def kernel(x, enc1_w, enc1_b, enc2_w, enc2_b, codebook,
           dec1_w, dec1_b, dec2_w, dec2_b):
    f32 = jnp.float32
    n = x.shape[0]

    xp = jnp.pad(x[:, 0], ((0, 0), (1, 1), (1, 1)))
    p16 = jnp.stack([xp[:, kh:kh + 224:2, kw:kw + 224:2]
                     for kh in range(4) for kw in range(4)], axis=-1)
    w1 = enc1_w[:, 0].reshape(_HID, 16).T
    z1 = pl.pallas_call(
        _enc1_body,
        grid=(n,),
        in_specs=[
            pl.BlockSpec((1, 112, 112, 16), lambda b: (b, 0, 0, 0)),
            pl.BlockSpec((16, _HID), lambda b: (0, 0)),
            pl.BlockSpec((1, _HID), lambda b: (0, 0)),
        ],
        out_specs=pl.BlockSpec((1, 112, 112, _HID), lambda b: (b, 0, 0, 0)),
        out_shape=jax.ShapeDtypeStruct((n, 112, 112, _HID), f32),
    )(p16, w1, enc1_b.reshape(1, _HID))

    z1p = jnp.pad(z1, ((0, 0), (1, 1), (1, 1), (0, 0)))
    e = (z1p.reshape(n, 57, 2, 57, 2, _HID)
         .transpose(0, 1, 3, 2, 4, 5)
         .reshape(n, 57, 57, 4 * _HID))
    w2 = jnp.stack([
        jnp.concatenate([enc2_w[:, :, a + 2 * i, b + 2 * j].T
                         for a in range(2) for b in range(2)], axis=0)
        for i in range(2) for j in range(2)], axis=0)
    z = pl.pallas_call(
        _enc2_body,
        grid=(n,),
        in_specs=[
            pl.BlockSpec((1, 57, 57, 4 * _HID), lambda b: (b, 0, 0, 0)),
            pl.BlockSpec((4, 4 * _HID, _EMB), lambda b: (0, 0, 0)),
            pl.BlockSpec((1, _EMB), lambda b: (0, 0)),
        ],
        out_specs=pl.BlockSpec((1, 56, 56, _EMB), lambda b: (b, 0, 0, 0)),
        out_shape=jax.ShapeDtypeStruct((n, 56, 56, _EMB), f32),
    )(e, w2, enc2_b.reshape(1, _EMB))

    nrows = n * 56 * 56
    zf3 = z.reshape(nrows // 1024, 1024, _EMB)
    idx3 = pl.pallas_call(
        _vq_body,
        grid=(nrows // 1024,),
        in_specs=[
            pl.BlockSpec((1, 1024, _EMB), lambda b: (b, 0, 0)),
            pl.BlockSpec((_EMB, _K), lambda b: (0, 0)),
        ],
        out_specs=pl.BlockSpec((1, 1, 1024), lambda b: (b, 0, 0)),
        out_shape=jax.ShapeDtypeStruct((nrows // 1024, 1, 1024), jnp.int32),
    )(zf3, codebook.T)
    idx_flat = idx3.reshape(nrows)

    q = _vq_gather(codebook, idx_flat)

    w3 = jnp.stack([dec1_w[:, :, 2 * i + py, 2 * j + px].T
                    for py in range(2) for px in range(2)
                    for i in range(2) for j in range(2)], axis=0)
    y_ph, ls = pl.pallas_call(
        _dec1_body,
        grid=(n,),
        in_specs=[
            pl.BlockSpec((1, 56, 56, _EMB), lambda b: (b, 0, 0, 0)),
            pl.BlockSpec((1, 56, 56, 128), lambda b: (b, 0, 0, 0)),
            pl.BlockSpec((16, _EMB, _HID), lambda b: (0, 0, 0)),
            pl.BlockSpec((1, _HID), lambda b: (0, 0)),
        ],
        out_specs=[
            pl.BlockSpec((4, 1, 56, 56, _HID), lambda b: (0, b, 0, 0, 0)),
            pl.BlockSpec(memory_space=pltpu.SMEM),
        ],
        out_shape=[
            jax.ShapeDtypeStruct((4, n, 56, 56, _HID), f32),
            jax.ShapeDtypeStruct((1, 1), f32),
        ],
    )(z, q.reshape(n, 56, 56, 128), w3, dec1_b.reshape(1, _HID))
    m = ls[0, 0] / float(nrows * _EMB)
    vq_loss = m + 0.25 * m
    y1 = (y_ph.reshape(2, 2, n, 56, 56, _HID)
          .transpose(2, 3, 0, 4, 1, 5)
          .reshape(n, 112, 112, _HID))

    y1p = jnp.pad(y1, ((0, 0), (1, 1), (1, 1), (0, 0)))
    w9cols = []
    for u in range(3):
        for v in range(3):
            col = []
            for py in range(2):
                for px in range(2):
                    i_, j_ = u - py, v - px
                    if 0 <= i_ <= 1 and 0 <= j_ <= 1:
                        col.append(dec2_w[0, :, 2 * i_ + py, 2 * j_ + px])
                    else:
                        col.append(jnp.zeros((_HID,), f32))
            w9cols.append(jnp.stack(col, axis=1))
    w9 = jnp.stack(w9cols, axis=0)
    r_ph = pl.pallas_call(
        _dec2_body,
        grid=(n,),
        in_specs=[
            pl.BlockSpec((1, 114, 114, _HID), lambda b: (b, 0, 0, 0)),
            pl.BlockSpec((9, _HID, 4), lambda b: (0, 0, 0)),
            pl.BlockSpec((1, 1), lambda b: (0, 0), memory_space=pltpu.SMEM),
        ],
        out_specs=pl.BlockSpec((1, 112, 112, 4), lambda b: (b, 0, 0, 0)),
        out_shape=jax.ShapeDtypeStruct((n, 112, 112, 4), f32),
    )(y1p, w9, dec2_b.reshape(1, 1))
    x_recon = (r_ph.reshape(n, 112, 112, 2, 2)
               .transpose(0, 1, 3, 2, 4)
               .reshape(n, 1, 224, 224))

    indices = idx_flat.reshape(n, 56, 56)
    return (x_recon, vq_loss, indices)
